# baseline (speedup 1.0000x reference)
import functools

import jax
import jax.numpy as jnp
from jax import lax
from jax.experimental import pallas as pl
from jax.experimental.pallas import tpu as pltpu
from jax.experimental.pallas import tpu_sc as plsc

E = 8
TK = 2
D = 2048
F = 768
T = 2048
P = T * TK
BM = 128
NB = P // BM + E
NQ = NB * BM

NW = 32
G_CH = 16
C_CH = 8
CS = 512


def _router_sched_body(x_ref, rw_ref, pos_ref, topw_ref, meta_ref):
    x = x_ref[...]
    rw = rw_ref[...]
    lt = lax.dot_general(rw, x, (((1,), (1,)), ((), ())),
                         preferred_element_type=jnp.float32)
    iota_e = lax.broadcasted_iota(jnp.int32, (E, T), 0)
    l1 = jnp.max(lt, axis=0, keepdims=True)
    i1 = jnp.min(jnp.where(lt == l1, iota_e, E), axis=0, keepdims=True)
    m0 = (iota_e == i1)
    ltm = jnp.where(m0, -jnp.inf, lt)
    l2 = jnp.max(ltm, axis=0, keepdims=True)
    i2 = jnp.min(jnp.where(ltm == l2, iota_e, E), axis=0, keepdims=True)
    m1 = (iota_e == i2)
    w0 = 1.0 / (1.0 + jnp.exp(l2 - l1))
    w1v = 1.0 - w0

    m = jnp.concatenate([m0, m1], axis=1).astype(jnp.float32)
    tri = (lax.broadcasted_iota(jnp.int32, (CS, CS), 0)
           <= lax.broadcasted_iota(jnp.int32, (CS, CS), 1)).astype(jnp.float32)
    chunks = []
    run = jnp.zeros((E, 1), jnp.float32)
    for c in range(P // CS):
        mc = lax.slice(m, (0, c * CS), (E, (c + 1) * CS))
        local = lax.dot_general(mc, tri, (((1,), (0,)), ((), ())),
                                preferred_element_type=jnp.float32)
        chunks.append(local + run)
        run = run + lax.slice(local, (0, CS - 1), (E, CS))
    cum = jnp.concatenate(chunks, axis=1)
    cnt = run
    pcnt = jnp.ceil(cnt * (1.0 / BM)) * BM
    low = (lax.broadcasted_iota(jnp.int32, (E, E), 0)
           > lax.broadcasted_iota(jnp.int32, (E, E), 1)).astype(jnp.float32)
    offs = lax.dot_general(low, pcnt, (((1,), (0,)), ((), ())),
                           preferred_element_type=jnp.float32)

    m0f = m0.astype(jnp.float32)
    m1f = m1.astype(jnp.float32)
    c0 = jnp.sum(m0f * lax.slice(cum, (0, 0), (E, T)), axis=0, keepdims=True)
    c1 = jnp.sum(m1f * lax.slice(cum, (0, T), (E, 2 * T)), axis=0, keepdims=True)
    o0 = jnp.sum(m0f * offs, axis=0, keepdims=True)
    o1 = jnp.sum(m1f * offs, axis=0, keepdims=True)
    pos0 = o0 + c0 - 1.0
    pos1 = o1 + c1 - 1.0
    pos_ref[...] = jnp.concatenate([pos0, pos1], axis=0).astype(jnp.int32)
    topw_ref[...] = jnp.concatenate([w0, w1v], axis=0)

    ends = offs + pcnt
    qs = lax.broadcasted_iota(jnp.int32, (1, NB), 1).astype(jnp.float32) * BM
    bexp = jnp.sum((ends <= qs).astype(jnp.float32), axis=0, keepdims=True)
    bexp = jnp.minimum(bexp, float(E - 1))
    nact = (jnp.sum(pcnt) * (1.0 / BM)).reshape(1, 1)
    meta_ref[...] = jnp.concatenate([nact, bexp], axis=1).astype(jnp.int32)


def _ffn_body(meta_ref, xg_ref, w1_ref, v1_ref, w2_ref, y_ref):
    b = pl.program_id(0)

    @pl.when(b < meta_ref[0])
    def _():
        xb = xg_ref[...]
        a = lax.dot_general(xb, w1_ref[0], (((1,), (1,)), ((), ())),
                            preferred_element_type=jnp.float32)
        u = lax.dot_general(xb, v1_ref[0], (((1,), (1,)), ((), ())),
                            preferred_element_type=jnp.float32)
        h = (a * jax.nn.sigmoid(a)) * u
        y_ref[...] = jnp.dot(h, w2_ref[0], preferred_element_type=jnp.float32)


def _sc_dispatch_body(x_hbm, pos_hbm, xg_hbm,
                      idx0, idx1, rows0, rows1, si0, si1, so0, so1):
    wid = lax.axis_index("s") * 2 + lax.axis_index("c")
    per_w = P // NW
    base = wid * per_w
    nch = per_w // G_CH
    idx = (idx0, idx1)
    rows = (rows0, rows1)
    sin = (si0, si1)
    sout = (so0, so1)

    def fill(c, b):
        p0 = base + c * G_CH
        t0 = p0 - (p0 // T) * T
        pltpu.sync_copy(pos_hbm.at[pl.ds(p0, G_CH)], idx[b])
        return pltpu.async_copy(x_hbm.at[pl.ds(t0, G_CH)], rows[b], sin[b])

    in_h = [fill(0, 0), None]
    out_h = [None, None]
    for c in range(nch):
        b = c % 2
        ob = (c + 1) % 2
        in_h[b].wait()
        if c + 1 < nch:
            if out_h[ob] is not None:
                out_h[ob].wait()
            in_h[ob] = fill(c + 1, ob)
        out_h[b] = pltpu.async_copy(rows[b], xg_hbm.at[idx[b]], sout[b])
    out_h[nch % 2].wait()
    out_h[(nch - 1) % 2].wait()


def _sc_combine_body(y_hbm, pos_hbm, w_hbm, out_hbm,
                     i0a, i1a, w0a, w1a, r0a, r1a, oa,
                     i0b, i1b, w0b, w1b, r0b, r1b, ob_,
                     sg0a, sg1a, sg0b, sg1b, ssta, sstb):
    wid = lax.axis_index("s") * 2 + lax.axis_index("c")
    per_w = T // NW
    base = wid * per_w
    nch = per_w // C_CH
    i0 = (i0a, i0b)
    i1 = (i1a, i1b)
    w0 = (w0a, w0b)
    w1 = (w1a, w1b)
    r0 = (r0a, r0b)
    r1 = (r1a, r1b)
    o = (oa, ob_)
    sg0 = (sg0a, sg0b)
    sg1 = (sg1a, sg1b)
    sst = (ssta, sstb)

    def start(c, b):
        t0 = base + c * C_CH
        pltpu.sync_copy(pos_hbm.at[pl.ds(t0, C_CH)], i0[b])
        pltpu.sync_copy(pos_hbm.at[pl.ds(T + t0, C_CH)], i1[b])
        pltpu.sync_copy(w_hbm.at[pl.ds(t0, C_CH)], w0[b].at[pl.ds(0, C_CH)])
        pltpu.sync_copy(w_hbm.at[pl.ds(T + t0, C_CH)], w1[b].at[pl.ds(0, C_CH)])
        return (pltpu.async_copy(y_hbm.at[i0[b]], r0[b], sg0[b]),
                pltpu.async_copy(y_hbm.at[i1[b]], r1[b], sg1[b]))

    gh = [start(0, 0), start(1, 1)]
    st_h = [None, None]
    for c in range(nch):
        b = c % 2
        gh[b][0].wait()
        gh[b][1].wait()
        if st_h[b] is not None:
            st_h[b].wait()

        def row(i, rcarry):
            a = w0[b][pl.ds(i, 16)][0]
            bb = w1[b][pl.ds(i, 16)][0]
            for j in range(D // 16):
                sl = pl.ds(j * 16, 16)
                o[b][i, sl] = a * r0[b][i, sl] + bb * r1[b][i, sl]
            return rcarry

        lax.fori_loop(0, C_CH, row, 0)
        st_h[b] = pltpu.async_copy(o[b], out_hbm.at[pl.ds(base + c * C_CH, C_CH)],
                                   sst[b])
        if c + 2 < nch:
            gh[b] = start(c + 2, b)
    st_h[0].wait()
    st_h[1].wait()


@functools.cache
def _get_sc_kernels():
    mesh = plsc.VectorSubcoreMesh(core_axis_name="c", subcore_axis_name="s")
    dispatch = pl.kernel(
        _sc_dispatch_body,
        out_type=jax.ShapeDtypeStruct((NQ, D), jnp.float32),
        mesh=mesh,
        scratch_types=[
            pltpu.VMEM((G_CH,), jnp.int32),
            pltpu.VMEM((G_CH,), jnp.int32),
            pltpu.VMEM((G_CH, D), jnp.float32),
            pltpu.VMEM((G_CH, D), jnp.float32),
            pltpu.SemaphoreType.DMA,
            pltpu.SemaphoreType.DMA,
            pltpu.SemaphoreType.DMA,
            pltpu.SemaphoreType.DMA,
        ],
    )
    combine = pl.kernel(
        _sc_combine_body,
        out_type=jax.ShapeDtypeStruct((T, D), jnp.float32),
        mesh=mesh,
        scratch_types=(
            2 * [
                pltpu.VMEM((C_CH,), jnp.int32),
                pltpu.VMEM((C_CH,), jnp.int32),
                pltpu.VMEM((C_CH + 16,), jnp.float32),
                pltpu.VMEM((C_CH + 16,), jnp.float32),
                pltpu.VMEM((C_CH, D), jnp.float32),
                pltpu.VMEM((C_CH, D), jnp.float32),
                pltpu.VMEM((C_CH, D), jnp.float32),
            ]
            + 6 * [pltpu.SemaphoreType.DMA]
        ),
    )
    return dispatch, combine

_router_sched = pl.pallas_call(
    _router_sched_body,
    out_shape=[
        jax.ShapeDtypeStruct((TK, T), jnp.int32),
        jax.ShapeDtypeStruct((TK, T), jnp.float32),
        jax.ShapeDtypeStruct((1, NB + 1), jnp.int32),
    ],
)

_ffn = pl.pallas_call(
    _ffn_body,
    grid_spec=pltpu.PrefetchScalarGridSpec(
        num_scalar_prefetch=1,
        grid=(NB,),
        in_specs=[
            pl.BlockSpec((BM, D), lambda b, m: (b, 0)),
            pl.BlockSpec((1, F, D), lambda b, m: (m[b + 1], 0, 0)),
            pl.BlockSpec((1, F, D), lambda b, m: (m[b + 1], 0, 0)),
            pl.BlockSpec((1, F, D), lambda b, m: (m[b + 1], 0, 0)),
        ],
        out_specs=pl.BlockSpec((BM, D), lambda b, m: (b, 0)),
    ),
    out_shape=jax.ShapeDtypeStruct((NQ, D), jnp.float32),
)


@jax.jit
def kernel(hidden_states, router_w, w1, v1, w2):
    B, S, Dh = hidden_states.shape
    x = hidden_states.reshape(T, D)
    dispatch, combine = _get_sc_kernels()
    pos2, topw2, meta2 = _router_sched(x, router_w)
    posf = pos2.reshape(P)
    wf = topw2.reshape(P)
    meta = meta2.reshape(NB + 1)
    xg = dispatch(x, posf)
    y = _ffn(meta, xg, w1, v1, w2)
    out = combine(y, posf, wf)
    return out.reshape(B, S, Dh)


# speedup vs baseline: 1.1866x; 1.1866x over previous
import functools

import jax
import jax.numpy as jnp
from jax import lax
from jax.experimental import pallas as pl
from jax.experimental.pallas import tpu as pltpu
from jax.experimental.pallas import tpu_sc as plsc

E = 8
TK = 2
D = 2048
F = 768
T = 2048
P = T * TK
BM = 256
NB = P // BM + E
NQ = NB * BM

NW = 32
G_CH = 16
C_CH = 8
CS = 512


def _router_sched_body(x_ref, rw_ref, pos_ref, topw_ref, meta_ref):
    x = x_ref[...]
    rw = rw_ref[...]
    lt = lax.dot_general(rw, x, (((1,), (1,)), ((), ())),
                         preferred_element_type=jnp.float32)
    iota_e = lax.broadcasted_iota(jnp.int32, (E, T), 0)
    l1 = jnp.max(lt, axis=0, keepdims=True)
    i1 = jnp.min(jnp.where(lt == l1, iota_e, E), axis=0, keepdims=True)
    m0 = (iota_e == i1)
    ltm = jnp.where(m0, -jnp.inf, lt)
    l2 = jnp.max(ltm, axis=0, keepdims=True)
    i2 = jnp.min(jnp.where(ltm == l2, iota_e, E), axis=0, keepdims=True)
    m1 = (iota_e == i2)
    w0 = 1.0 / (1.0 + jnp.exp(l2 - l1))
    w1v = 1.0 - w0

    m = jnp.concatenate([m0, m1], axis=1).astype(jnp.float32)
    tri = (lax.broadcasted_iota(jnp.int32, (CS, CS), 0)
           <= lax.broadcasted_iota(jnp.int32, (CS, CS), 1)).astype(jnp.float32)
    chunks = []
    run = jnp.zeros((E, 1), jnp.float32)
    for c in range(P // CS):
        mc = lax.slice(m, (0, c * CS), (E, (c + 1) * CS))
        local = lax.dot_general(mc, tri, (((1,), (0,)), ((), ())),
                                preferred_element_type=jnp.float32)
        chunks.append(local + run)
        run = run + lax.slice(local, (0, CS - 1), (E, CS))
    cum = jnp.concatenate(chunks, axis=1)
    cnt = run
    pcnt = jnp.ceil(cnt * (1.0 / BM)) * BM
    low = (lax.broadcasted_iota(jnp.int32, (E, E), 0)
           > lax.broadcasted_iota(jnp.int32, (E, E), 1)).astype(jnp.float32)
    offs = lax.dot_general(low, pcnt, (((1,), (0,)), ((), ())),
                           preferred_element_type=jnp.float32)

    m0f = m0.astype(jnp.float32)
    m1f = m1.astype(jnp.float32)
    c0 = jnp.sum(m0f * lax.slice(cum, (0, 0), (E, T)), axis=0, keepdims=True)
    c1 = jnp.sum(m1f * lax.slice(cum, (0, T), (E, 2 * T)), axis=0, keepdims=True)
    o0 = jnp.sum(m0f * offs, axis=0, keepdims=True)
    o1 = jnp.sum(m1f * offs, axis=0, keepdims=True)
    pos0 = o0 + c0 - 1.0
    pos1 = o1 + c1 - 1.0
    pos_ref[...] = jnp.concatenate([pos0, pos1], axis=0).astype(jnp.int32)
    topw_ref[...] = jnp.concatenate([w0, w1v], axis=0)

    ends = offs + pcnt
    qs = lax.broadcasted_iota(jnp.int32, (1, NB), 1).astype(jnp.float32) * BM
    bexp = jnp.sum((ends <= qs).astype(jnp.float32), axis=0, keepdims=True)
    bexp = jnp.minimum(bexp, float(E - 1))
    nact = (jnp.sum(pcnt) * (1.0 / BM)).reshape(1, 1)
    meta_ref[...] = jnp.concatenate([nact, bexp], axis=1).astype(jnp.int32)


def _ffn_body(meta_ref, xg_ref, w1_ref, v1_ref, w2_ref, y_ref):
    b = pl.program_id(0)

    @pl.when(b < meta_ref[0])
    def _():
        xb = xg_ref[...]
        a = lax.dot_general(xb, w1_ref[0], (((1,), (1,)), ((), ())),
                            preferred_element_type=jnp.float32)
        u = lax.dot_general(xb, v1_ref[0], (((1,), (1,)), ((), ())),
                            preferred_element_type=jnp.float32)
        h = (a * jax.nn.sigmoid(a)) * u
        y_ref[...] = jnp.dot(h, w2_ref[0], preferred_element_type=jnp.float32)


def _sc_dispatch_body(x_hbm, pos_hbm, xg_hbm,
                      idx0, idx1, rows0, rows1, si0, si1, so0, so1):
    wid = lax.axis_index("s") * 2 + lax.axis_index("c")
    per_w = P // NW
    base = wid * per_w
    nch = per_w // G_CH
    idx = (idx0, idx1)
    rows = (rows0, rows1)
    sin = (si0, si1)
    sout = (so0, so1)

    def fill(c, b):
        p0 = base + c * G_CH
        t0 = p0 - (p0 // T) * T
        pltpu.sync_copy(pos_hbm.at[pl.ds(p0, G_CH)], idx[b])
        return pltpu.async_copy(x_hbm.at[pl.ds(t0, G_CH)], rows[b], sin[b])

    in_h = [fill(0, 0), None]
    out_h = [None, None]
    for c in range(nch):
        b = c % 2
        ob = (c + 1) % 2
        in_h[b].wait()
        if c + 1 < nch:
            if out_h[ob] is not None:
                out_h[ob].wait()
            in_h[ob] = fill(c + 1, ob)
        out_h[b] = pltpu.async_copy(rows[b], xg_hbm.at[idx[b]], sout[b])
    out_h[nch % 2].wait()
    out_h[(nch - 1) % 2].wait()


def _sc_combine_body(y_hbm, pos_hbm, w_hbm, out_hbm,
                     i0a, i1a, w0a, w1a, r0a, r1a, oa,
                     i0b, i1b, w0b, w1b, r0b, r1b, ob_,
                     sg0a, sg1a, sg0b, sg1b, ssta, sstb):
    wid = lax.axis_index("s") * 2 + lax.axis_index("c")
    per_w = T // NW
    base = wid * per_w
    nch = per_w // C_CH
    i0 = (i0a, i0b)
    i1 = (i1a, i1b)
    w0 = (w0a, w0b)
    w1 = (w1a, w1b)
    r0 = (r0a, r0b)
    r1 = (r1a, r1b)
    o = (oa, ob_)
    sg0 = (sg0a, sg0b)
    sg1 = (sg1a, sg1b)
    sst = (ssta, sstb)

    def start(c, b):
        t0 = base + c * C_CH
        pltpu.sync_copy(pos_hbm.at[pl.ds(t0, C_CH)], i0[b])
        pltpu.sync_copy(pos_hbm.at[pl.ds(T + t0, C_CH)], i1[b])
        pltpu.sync_copy(w_hbm.at[pl.ds(t0, C_CH)], w0[b].at[pl.ds(0, C_CH)])
        pltpu.sync_copy(w_hbm.at[pl.ds(T + t0, C_CH)], w1[b].at[pl.ds(0, C_CH)])
        return (pltpu.async_copy(y_hbm.at[i0[b]], r0[b], sg0[b]),
                pltpu.async_copy(y_hbm.at[i1[b]], r1[b], sg1[b]))

    gh = [start(0, 0), start(1, 1)]
    st_h = [None, None]
    for c in range(nch):
        b = c % 2
        gh[b][0].wait()
        gh[b][1].wait()
        if st_h[b] is not None:
            st_h[b].wait()

        def row(i, rcarry):
            a = w0[b][pl.ds(i, 16)][0]
            bb = w1[b][pl.ds(i, 16)][0]
            for j in range(D // 16):
                sl = pl.ds(j * 16, 16)
                o[b][i, sl] = a * r0[b][i, sl] + bb * r1[b][i, sl]
            return rcarry

        lax.fori_loop(0, C_CH, row, 0)
        st_h[b] = pltpu.async_copy(o[b], out_hbm.at[pl.ds(base + c * C_CH, C_CH)],
                                   sst[b])
        if c + 2 < nch:
            gh[b] = start(c + 2, b)
    st_h[0].wait()
    st_h[1].wait()


@functools.cache
def _get_sc_kernels():
    mesh = plsc.VectorSubcoreMesh(core_axis_name="c", subcore_axis_name="s")
    dispatch = pl.kernel(
        _sc_dispatch_body,
        out_type=jax.ShapeDtypeStruct((NQ, D), jnp.float32),
        mesh=mesh,
        scratch_types=[
            pltpu.VMEM((G_CH,), jnp.int32),
            pltpu.VMEM((G_CH,), jnp.int32),
            pltpu.VMEM((G_CH, D), jnp.float32),
            pltpu.VMEM((G_CH, D), jnp.float32),
            pltpu.SemaphoreType.DMA,
            pltpu.SemaphoreType.DMA,
            pltpu.SemaphoreType.DMA,
            pltpu.SemaphoreType.DMA,
        ],
    )
    combine = pl.kernel(
        _sc_combine_body,
        out_type=jax.ShapeDtypeStruct((T, D), jnp.float32),
        mesh=mesh,
        scratch_types=(
            2 * [
                pltpu.VMEM((C_CH,), jnp.int32),
                pltpu.VMEM((C_CH,), jnp.int32),
                pltpu.VMEM((C_CH + 16,), jnp.float32),
                pltpu.VMEM((C_CH + 16,), jnp.float32),
                pltpu.VMEM((C_CH, D), jnp.float32),
                pltpu.VMEM((C_CH, D), jnp.float32),
                pltpu.VMEM((C_CH, D), jnp.float32),
            ]
            + 6 * [pltpu.SemaphoreType.DMA]
        ),
    )
    return dispatch, combine

_router_sched = pl.pallas_call(
    _router_sched_body,
    out_shape=[
        jax.ShapeDtypeStruct((TK, T), jnp.int32),
        jax.ShapeDtypeStruct((TK, T), jnp.float32),
        jax.ShapeDtypeStruct((1, NB + 1), jnp.int32),
    ],
)

_ffn = pl.pallas_call(
    _ffn_body,
    grid_spec=pltpu.PrefetchScalarGridSpec(
        num_scalar_prefetch=1,
        grid=(NB,),
        in_specs=[
            pl.BlockSpec((BM, D), lambda b, m: (b, 0)),
            pl.BlockSpec((1, F, D), lambda b, m: (m[b + 1], 0, 0)),
            pl.BlockSpec((1, F, D), lambda b, m: (m[b + 1], 0, 0)),
            pl.BlockSpec((1, F, D), lambda b, m: (m[b + 1], 0, 0)),
        ],
        out_specs=pl.BlockSpec((BM, D), lambda b, m: (b, 0)),
    ),
    out_shape=jax.ShapeDtypeStruct((NQ, D), jnp.float32),
)


@jax.jit
def kernel(hidden_states, router_w, w1, v1, w2):
    B, S, Dh = hidden_states.shape
    x = hidden_states.reshape(T, D)
    dispatch, combine = _get_sc_kernels()
    pos2, topw2, meta2 = _router_sched(x, router_w)
    posf = pos2.reshape(P)
    wf = topw2.reshape(P)
    meta = meta2.reshape(NB + 1)
    xg = dispatch(x, posf)
    y = _ffn(meta, xg, w1, v1, w2)
    out = combine(y, posf, wf)
    return out.reshape(B, S, Dh)


# speedup vs baseline: 1.2617x; 1.0633x over previous
import functools

import jax
import jax.numpy as jnp
from jax import lax
from jax.experimental import pallas as pl
from jax.experimental.pallas import tpu as pltpu
from jax.experimental.pallas import tpu_sc as plsc

E = 8
TK = 2
D = 2048
F = 768
T = 2048
P = T * TK
BM = 256
NB = P // BM + E
NQ = NB * BM

NW = 32
G_CH = 16
C_CH = 8
CS = 512


def _router_sched_body(x_ref, rw_ref, pos_ref, topw_ref, meta_ref):
    x = x_ref[...]
    rw = rw_ref[...]
    lt = lax.dot_general(rw, x, (((1,), (1,)), ((), ())),
                         preferred_element_type=jnp.float32)
    iota_e = lax.broadcasted_iota(jnp.int32, (E, T), 0)
    l1 = jnp.max(lt, axis=0, keepdims=True)
    i1 = jnp.min(jnp.where(lt == l1, iota_e, E), axis=0, keepdims=True)
    m0 = (iota_e == i1)
    ltm = jnp.where(m0, -jnp.inf, lt)
    l2 = jnp.max(ltm, axis=0, keepdims=True)
    i2 = jnp.min(jnp.where(ltm == l2, iota_e, E), axis=0, keepdims=True)
    m1 = (iota_e == i2)
    w0 = 1.0 / (1.0 + jnp.exp(l2 - l1))
    w1v = 1.0 - w0

    m = jnp.concatenate([m0, m1], axis=1).astype(jnp.float32)
    tri = (lax.broadcasted_iota(jnp.int32, (CS, CS), 0)
           <= lax.broadcasted_iota(jnp.int32, (CS, CS), 1)).astype(jnp.float32)
    chunks = []
    run = jnp.zeros((E, 1), jnp.float32)
    for c in range(P // CS):
        mc = lax.slice(m, (0, c * CS), (E, (c + 1) * CS))
        local = lax.dot_general(mc, tri, (((1,), (0,)), ((), ())),
                                preferred_element_type=jnp.float32)
        chunks.append(local + run)
        run = run + lax.slice(local, (0, CS - 1), (E, CS))
    cum = jnp.concatenate(chunks, axis=1)
    cnt = run
    pcnt = jnp.ceil(cnt * (1.0 / BM)) * BM
    low = (lax.broadcasted_iota(jnp.int32, (E, E), 0)
           > lax.broadcasted_iota(jnp.int32, (E, E), 1)).astype(jnp.float32)
    offs = lax.dot_general(low, pcnt, (((1,), (0,)), ((), ())),
                           preferred_element_type=jnp.float32)

    m0f = m0.astype(jnp.float32)
    m1f = m1.astype(jnp.float32)
    c0 = jnp.sum(m0f * lax.slice(cum, (0, 0), (E, T)), axis=0, keepdims=True)
    c1 = jnp.sum(m1f * lax.slice(cum, (0, T), (E, 2 * T)), axis=0, keepdims=True)
    o0 = jnp.sum(m0f * offs, axis=0, keepdims=True)
    o1 = jnp.sum(m1f * offs, axis=0, keepdims=True)
    pos0 = o0 + c0 - 1.0
    pos1 = o1 + c1 - 1.0
    pos_ref[...] = jnp.concatenate([pos0, pos1], axis=0).astype(jnp.int32)
    topw_ref[...] = jnp.concatenate([w0, w1v], axis=0)

    ends = offs + pcnt
    qs = lax.broadcasted_iota(jnp.int32, (1, NB), 1).astype(jnp.float32) * BM
    bexp = jnp.sum((ends <= qs).astype(jnp.float32), axis=0, keepdims=True)
    bexp = jnp.minimum(bexp, float(E - 1))
    nact = (jnp.sum(pcnt) * (1.0 / BM)).reshape(1, 1)
    meta_ref[...] = jnp.concatenate([nact, bexp], axis=1).astype(jnp.int32)


def _ffn_body(meta_ref, xg_ref, w1_ref, v1_ref, w2_ref, y_ref):
    b = pl.program_id(0)

    @pl.when(b < meta_ref[0])
    def _():
        xb = xg_ref[...]
        a = lax.dot_general(xb, w1_ref[0], (((1,), (1,)), ((), ())),
                            preferred_element_type=jnp.float32)
        u = lax.dot_general(xb, v1_ref[0], (((1,), (1,)), ((), ())),
                            preferred_element_type=jnp.float32)
        h = (a * jax.nn.sigmoid(a)) * u
        y_ref[...] = jnp.dot(h, w2_ref[0], preferred_element_type=jnp.float32)


def _sc_dispatch_body(x_hbm, pos2d_hbm, xg_hbm,
                      idx_all, rows0, rows1, si0, si1, so0, so1):
    wid = lax.axis_index("s") * 2 + lax.axis_index("c")
    per_w = P // NW
    base = wid * per_w
    nch = per_w // G_CH
    rows = (rows0, rows1)
    sin = (si0, si1)
    sout = (so0, so1)
    pltpu.sync_copy(pos2d_hbm.at[pl.ds(wid * nch, nch)], idx_all)

    def fill(c, b):
        p0 = base + c * G_CH
        t0 = p0 - (p0 // T) * T
        return pltpu.async_copy(x_hbm.at[pl.ds(t0, G_CH)], rows[b], sin[b])

    in_h = [fill(0, 0), None]
    out_h = [None, None]
    for c in range(nch):
        b = c % 2
        ob = (c + 1) % 2
        in_h[b].wait()
        if c + 1 < nch:
            if out_h[ob] is not None:
                out_h[ob].wait()
            in_h[ob] = fill(c + 1, ob)
        out_h[b] = pltpu.async_copy(rows[b], xg_hbm.at[idx_all.at[c]], sout[b])
    out_h[nch % 2].wait()
    out_h[(nch - 1) % 2].wait()


def _sc_combine_body(y_hbm, pos2d_hbm, w_hbm, out_hbm,
                     i0_all, i1_all, w0_all, w1_all,
                     r0a, r1a, oa, r0b, r1b, ob_,
                     sg0a, sg1a, sg0b, sg1b, ssta, sstb):
    wid = lax.axis_index("s") * 2 + lax.axis_index("c")
    per_w = T // NW
    base = wid * per_w
    nch = per_w // C_CH
    r0 = (r0a, r0b)
    r1 = (r1a, r1b)
    o = (oa, ob_)
    sg0 = (sg0a, sg0b)
    sg1 = (sg1a, sg1b)
    sst = (ssta, sstb)
    pltpu.sync_copy(pos2d_hbm.at[pl.ds(wid * nch, nch)], i0_all)
    pltpu.sync_copy(pos2d_hbm.at[pl.ds((T // C_CH) + wid * nch, nch)], i1_all)
    pltpu.sync_copy(w_hbm.at[pl.ds(base, per_w)], w0_all.at[pl.ds(0, per_w)])
    pltpu.sync_copy(w_hbm.at[pl.ds(T + base, per_w)], w1_all.at[pl.ds(0, per_w)])

    def start(c, b):
        return (pltpu.async_copy(y_hbm.at[i0_all.at[c]], r0[b], sg0[b]),
                pltpu.async_copy(y_hbm.at[i1_all.at[c]], r1[b], sg1[b]))

    gh = [start(0, 0), start(1, 1)]
    st_h = [None, None]
    for c in range(nch):
        b = c % 2
        gh[b][0].wait()
        gh[b][1].wait()
        if st_h[b] is not None:
            st_h[b].wait()

        def row(i, rcarry):
            a = w0_all[pl.ds(c * C_CH + i, 16)][0]
            bb = w1_all[pl.ds(c * C_CH + i, 16)][0]
            for j in range(D // 16):
                sl = pl.ds(j * 16, 16)
                o[b][i, sl] = a * r0[b][i, sl] + bb * r1[b][i, sl]
            return rcarry

        lax.fori_loop(0, C_CH, row, 0)
        st_h[b] = pltpu.async_copy(o[b], out_hbm.at[pl.ds(base + c * C_CH, C_CH)],
                                   sst[b])
        if c + 2 < nch:
            gh[b] = start(c + 2, b)
    st_h[0].wait()
    st_h[1].wait()


@functools.cache
def _get_sc_kernels():
    mesh = plsc.VectorSubcoreMesh(core_axis_name="c", subcore_axis_name="s")
    dispatch = pl.kernel(
        _sc_dispatch_body,
        out_type=jax.ShapeDtypeStruct((NQ, D), jnp.float32),
        mesh=mesh,
        scratch_types=[
            pltpu.VMEM((P // NW // G_CH, G_CH), jnp.int32),
            pltpu.VMEM((G_CH, D), jnp.float32),
            pltpu.VMEM((G_CH, D), jnp.float32),
            pltpu.SemaphoreType.DMA,
            pltpu.SemaphoreType.DMA,
            pltpu.SemaphoreType.DMA,
            pltpu.SemaphoreType.DMA,
        ],
    )
    combine = pl.kernel(
        _sc_combine_body,
        out_type=jax.ShapeDtypeStruct((T, D), jnp.float32),
        mesh=mesh,
        scratch_types=(
            [
                pltpu.VMEM((T // NW // C_CH, C_CH), jnp.int32),
                pltpu.VMEM((T // NW // C_CH, C_CH), jnp.int32),
                pltpu.VMEM((T // NW + 16,), jnp.float32),
                pltpu.VMEM((T // NW + 16,), jnp.float32),
            ]
            + 2 * [
                pltpu.VMEM((C_CH, D), jnp.float32),
                pltpu.VMEM((C_CH, D), jnp.float32),
                pltpu.VMEM((C_CH, D), jnp.float32),
            ]
            + 6 * [pltpu.SemaphoreType.DMA]
        ),
    )
    return dispatch, combine

_router_sched = pl.pallas_call(
    _router_sched_body,
    out_shape=[
        jax.ShapeDtypeStruct((TK, T), jnp.int32),
        jax.ShapeDtypeStruct((TK, T), jnp.float32),
        jax.ShapeDtypeStruct((1, NB + 1), jnp.int32),
    ],
)

_ffn = pl.pallas_call(
    _ffn_body,
    grid_spec=pltpu.PrefetchScalarGridSpec(
        num_scalar_prefetch=1,
        grid=(NB,),
        in_specs=[
            pl.BlockSpec((BM, D), lambda b, m: (b, 0)),
            pl.BlockSpec((1, F, D), lambda b, m: (m[b + 1], 0, 0)),
            pl.BlockSpec((1, F, D), lambda b, m: (m[b + 1], 0, 0)),
            pl.BlockSpec((1, F, D), lambda b, m: (m[b + 1], 0, 0)),
        ],
        out_specs=pl.BlockSpec((BM, D), lambda b, m: (b, 0)),
    ),
    out_shape=jax.ShapeDtypeStruct((NQ, D), jnp.float32),
)


@jax.jit
def kernel(hidden_states, router_w, w1, v1, w2):
    B, S, Dh = hidden_states.shape
    x = hidden_states.reshape(T, D)
    dispatch, combine = _get_sc_kernels()
    pos2, topw2, meta2 = _router_sched(x, router_w)
    posf = pos2.reshape(P)
    wf = topw2.reshape(P)
    meta = meta2.reshape(NB + 1)
    xg = dispatch(x, posf.reshape(P // G_CH, G_CH))
    y = _ffn(meta, xg, w1, v1, w2)
    out = combine(y, posf.reshape(P // C_CH, C_CH), wf)
    return out.reshape(B, S, Dh)


# speedup vs baseline: 1.3334x; 1.0568x over previous
import functools

import jax
import jax.numpy as jnp
from jax import lax
from jax.experimental import pallas as pl
from jax.experimental.pallas import tpu as pltpu
from jax.experimental.pallas import tpu_sc as plsc

E = 8
TK = 2
D = 2048
F = 768
T = 2048
P = T * TK
BM = 256
NB = P // BM + E
NQ = NB * BM

NW = 32
G_CH = 16
C_CH = 8
CS = 512


def _router_sched_body(x_ref, rw_ref, pos_ref, topw_ref, meta_ref, xbf_ref):
    x = x_ref[...]
    rw = rw_ref[...]
    lt = lax.dot_general(rw, x, (((1,), (1,)), ((), ())),
                         preferred_element_type=jnp.float32)
    iota_e = lax.broadcasted_iota(jnp.int32, (E, T), 0)
    l1 = jnp.max(lt, axis=0, keepdims=True)
    i1 = jnp.min(jnp.where(lt == l1, iota_e, E), axis=0, keepdims=True)
    m0 = (iota_e == i1)
    ltm = jnp.where(m0, -jnp.inf, lt)
    l2 = jnp.max(ltm, axis=0, keepdims=True)
    i2 = jnp.min(jnp.where(ltm == l2, iota_e, E), axis=0, keepdims=True)
    m1 = (iota_e == i2)
    w0 = 1.0 / (1.0 + jnp.exp(l2 - l1))
    w1v = 1.0 - w0

    m = jnp.concatenate([m0, m1], axis=1).astype(jnp.float32)
    tri = (lax.broadcasted_iota(jnp.int32, (CS, CS), 0)
           <= lax.broadcasted_iota(jnp.int32, (CS, CS), 1)).astype(jnp.float32)
    chunks = []
    run = jnp.zeros((E, 1), jnp.float32)
    for c in range(P // CS):
        mc = lax.slice(m, (0, c * CS), (E, (c + 1) * CS))
        local = lax.dot_general(mc, tri, (((1,), (0,)), ((), ())),
                                preferred_element_type=jnp.float32)
        chunks.append(local + run)
        run = run + lax.slice(local, (0, CS - 1), (E, CS))
    cum = jnp.concatenate(chunks, axis=1)
    cnt = run
    pcnt = jnp.ceil(cnt * (1.0 / BM)) * BM
    low = (lax.broadcasted_iota(jnp.int32, (E, E), 0)
           > lax.broadcasted_iota(jnp.int32, (E, E), 1)).astype(jnp.float32)
    offs = lax.dot_general(low, pcnt, (((1,), (0,)), ((), ())),
                           preferred_element_type=jnp.float32)

    m0f = m0.astype(jnp.float32)
    m1f = m1.astype(jnp.float32)
    c0 = jnp.sum(m0f * lax.slice(cum, (0, 0), (E, T)), axis=0, keepdims=True)
    c1 = jnp.sum(m1f * lax.slice(cum, (0, T), (E, 2 * T)), axis=0, keepdims=True)
    o0 = jnp.sum(m0f * offs, axis=0, keepdims=True)
    o1 = jnp.sum(m1f * offs, axis=0, keepdims=True)
    pos0 = o0 + c0 - 1.0
    pos1 = o1 + c1 - 1.0
    pos_ref[...] = jnp.concatenate([pos0, pos1], axis=0).astype(jnp.int32)
    topw_ref[...] = jnp.concatenate([w0, w1v], axis=0)

    ends = offs + pcnt
    qs = lax.broadcasted_iota(jnp.int32, (1, NB), 1).astype(jnp.float32) * BM
    bexp = jnp.sum((ends <= qs).astype(jnp.float32), axis=0, keepdims=True)
    bexp = jnp.minimum(bexp, float(E - 1))
    nact = (jnp.sum(pcnt) * (1.0 / BM)).reshape(1, 1)
    meta_ref[...] = jnp.concatenate([nact, bexp], axis=1).astype(jnp.int32)
    rb_lo = lax.slice(x, (0, 0), (T, D // 2)).astype(jnp.bfloat16).astype(jnp.float32)
    rb_hi = lax.slice(x, (0, D // 2), (T, D)).astype(jnp.bfloat16).astype(jnp.float32)
    i_lo = lax.shift_right_logical(lax.bitcast_convert_type(rb_lo, jnp.int32), 16)
    i_hi = jnp.bitwise_and(lax.bitcast_convert_type(rb_hi, jnp.int32),
                           jnp.int32(-65536))
    xbf_ref[...] = lax.bitcast_convert_type(i_lo | i_hi, jnp.float32)


def _ffn_body(meta_ref, xg_ref, w1_ref, v1_ref, w2_ref, y_ref):
    b = pl.program_id(0)

    @pl.when(b < meta_ref[0])
    def _():
        w_i = lax.bitcast_convert_type(xg_ref[...], jnp.int32)
        lo = lax.bitcast_convert_type(lax.shift_left(w_i, 16), jnp.float32)
        hi = lax.bitcast_convert_type(
            jnp.bitwise_and(w_i, jnp.int32(-65536)), jnp.float32)
        xb = jnp.concatenate([lo, hi], axis=1)
        a = lax.dot_general(xb, w1_ref[0], (((1,), (1,)), ((), ())),
                            preferred_element_type=jnp.float32)
        u = lax.dot_general(xb, v1_ref[0], (((1,), (1,)), ((), ())),
                            preferred_element_type=jnp.float32)
        h = (a * jax.nn.sigmoid(a)) * u
        y_ref[...] = jnp.dot(h, w2_ref[0], preferred_element_type=jnp.float32)


def _sc_dispatch_body(x_hbm, pos2d_hbm, xg_hbm,
                      idx_all, rows0, rows1, si0, si1, so0, so1):
    wid = lax.axis_index("s") * 2 + lax.axis_index("c")
    per_w = P // NW
    base = wid * per_w
    nch = per_w // G_CH
    rows = (rows0, rows1)
    sin = (si0, si1)
    sout = (so0, so1)
    pltpu.sync_copy(pos2d_hbm.at[pl.ds(wid * nch, nch)], idx_all)

    def fill(c, b):
        p0 = base + c * G_CH
        t0 = p0 - (p0 // T) * T
        return pltpu.async_copy(x_hbm.at[pl.ds(t0, G_CH)], rows[b], sin[b])

    in_h = [fill(0, 0), None]
    out_h = [None, None]
    for c in range(nch):
        b = c % 2
        ob = (c + 1) % 2
        in_h[b].wait()
        if c + 1 < nch:
            if out_h[ob] is not None:
                out_h[ob].wait()
            in_h[ob] = fill(c + 1, ob)
        out_h[b] = pltpu.async_copy(rows[b], xg_hbm.at[idx_all.at[c]], sout[b])
    out_h[nch % 2].wait()
    out_h[(nch - 1) % 2].wait()


def _sc_combine_body(y_hbm, pos2d_hbm, w_hbm, out_hbm,
                     i0_all, i1_all, w0_all, w1_all,
                     r0a, r1a, oa, r0b, r1b, ob_,
                     sg0a, sg1a, sg0b, sg1b, ssta, sstb):
    wid = lax.axis_index("s") * 2 + lax.axis_index("c")
    per_w = T // NW
    base = wid * per_w
    nch = per_w // C_CH
    r0 = (r0a, r0b)
    r1 = (r1a, r1b)
    o = (oa, ob_)
    sg0 = (sg0a, sg0b)
    sg1 = (sg1a, sg1b)
    sst = (ssta, sstb)
    pltpu.sync_copy(pos2d_hbm.at[pl.ds(wid * nch, nch)], i0_all)
    pltpu.sync_copy(pos2d_hbm.at[pl.ds((T // C_CH) + wid * nch, nch)], i1_all)
    pltpu.sync_copy(w_hbm.at[pl.ds(base, per_w)], w0_all.at[pl.ds(0, per_w)])
    pltpu.sync_copy(w_hbm.at[pl.ds(T + base, per_w)], w1_all.at[pl.ds(0, per_w)])

    def start(c, b):
        return (pltpu.async_copy(y_hbm.at[i0_all.at[c]], r0[b], sg0[b]),
                pltpu.async_copy(y_hbm.at[i1_all.at[c]], r1[b], sg1[b]))

    gh = [start(0, 0), start(1, 1)]
    st_h = [None, None]
    for c in range(nch):
        b = c % 2
        gh[b][0].wait()
        gh[b][1].wait()
        if st_h[b] is not None:
            st_h[b].wait()

        def row(i, rcarry):
            a = w0_all[pl.ds(c * C_CH + i, 16)][0]
            bb = w1_all[pl.ds(c * C_CH + i, 16)][0]
            for j in range(D // 16):
                sl = pl.ds(j * 16, 16)
                o[b][i, sl] = a * r0[b][i, sl] + bb * r1[b][i, sl]
            return rcarry

        lax.fori_loop(0, C_CH, row, 0)
        st_h[b] = pltpu.async_copy(o[b], out_hbm.at[pl.ds(base + c * C_CH, C_CH)],
                                   sst[b])
        if c + 2 < nch:
            gh[b] = start(c + 2, b)
    st_h[0].wait()
    st_h[1].wait()


@functools.cache
def _get_sc_kernels():
    mesh = plsc.VectorSubcoreMesh(core_axis_name="c", subcore_axis_name="s")
    dispatch = pl.kernel(
        _sc_dispatch_body,
        out_type=jax.ShapeDtypeStruct((NQ, D // 2), jnp.float32),
        mesh=mesh,
        scratch_types=[
            pltpu.VMEM((P // NW // G_CH, G_CH), jnp.int32),
            pltpu.VMEM((G_CH, D // 2), jnp.float32),
            pltpu.VMEM((G_CH, D // 2), jnp.float32),
            pltpu.SemaphoreType.DMA,
            pltpu.SemaphoreType.DMA,
            pltpu.SemaphoreType.DMA,
            pltpu.SemaphoreType.DMA,
        ],
    )
    combine = pl.kernel(
        _sc_combine_body,
        out_type=jax.ShapeDtypeStruct((T, D), jnp.float32),
        mesh=mesh,
        scratch_types=(
            [
                pltpu.VMEM((T // NW // C_CH, C_CH), jnp.int32),
                pltpu.VMEM((T // NW // C_CH, C_CH), jnp.int32),
                pltpu.VMEM((T // NW + 16,), jnp.float32),
                pltpu.VMEM((T // NW + 16,), jnp.float32),
            ]
            + 2 * [
                pltpu.VMEM((C_CH, D), jnp.float32),
                pltpu.VMEM((C_CH, D), jnp.float32),
                pltpu.VMEM((C_CH, D), jnp.float32),
            ]
            + 6 * [pltpu.SemaphoreType.DMA]
        ),
    )
    return dispatch, combine

_router_sched = pl.pallas_call(
    _router_sched_body,
    out_shape=[
        jax.ShapeDtypeStruct((TK, T), jnp.int32),
        jax.ShapeDtypeStruct((TK, T), jnp.float32),
        jax.ShapeDtypeStruct((1, NB + 1), jnp.int32),
        jax.ShapeDtypeStruct((T, D // 2), jnp.float32),
    ],
)

_ffn = pl.pallas_call(
    _ffn_body,
    grid_spec=pltpu.PrefetchScalarGridSpec(
        num_scalar_prefetch=1,
        grid=(NB,),
        in_specs=[
            pl.BlockSpec((BM, D // 2), lambda b, m: (b, 0)),
            pl.BlockSpec((1, F, D), lambda b, m: (m[b + 1], 0, 0)),
            pl.BlockSpec((1, F, D), lambda b, m: (m[b + 1], 0, 0)),
            pl.BlockSpec((1, F, D), lambda b, m: (m[b + 1], 0, 0)),
        ],
        out_specs=pl.BlockSpec((BM, D), lambda b, m: (b, 0)),
    ),
    out_shape=jax.ShapeDtypeStruct((NQ, D), jnp.float32),
)


@jax.jit
def kernel(hidden_states, router_w, w1, v1, w2):
    B, S, Dh = hidden_states.shape
    x = hidden_states.reshape(T, D)
    dispatch, combine = _get_sc_kernels()
    pos2, topw2, meta2, xbf = _router_sched(x, router_w)
    posf = pos2.reshape(P)
    wf = topw2.reshape(P)
    meta = meta2.reshape(NB + 1)
    xg = dispatch(xbf, posf.reshape(P // G_CH, G_CH))
    y = _ffn(meta, xg, w1, v1, w2)
    out = combine(y, posf.reshape(P // C_CH, C_CH), wf)
    return out.reshape(B, S, Dh)
